# baseline (speedup 1.0000x reference)
import jax
import jax.numpy as jnp
from jax import lax
from jax.experimental import pallas as pl
from jax.experimental.pallas import tpu as pltpu
from jax.experimental.pallas import tpu_sc as plsc

N_NODES_C = 10000
N_EDGES_C = 320000
D = 128
DW = D // 2
ACCP = 17

NC = 2
NS = 16
NW = NC * NS
EW = N_EDGES_C // NW
SUPER = 128
GCH = 128
NSUPER = 80
LAST_BASE = EW - SUPER


def _fire(hp_hbm, idx, rows, base_l, sem):
    for k in range(SUPER // GCH):
        pltpu.async_copy(
            hp_hbm.at[idx.at[pl.ds(base_l + k * GCH, GCH)]],
            rows.at[pl.ds(k * GCH, GCH)],
            sem,
        )


def _drain(hp_hbm, idx, rows, sem):
    for k in range(SUPER // GCH):
        pltpu.make_async_copy(
            hp_hbm.at[idx.at[pl.ds(k * GCH, GCH)]],
            rows.at[pl.ds(k * GCH, GCH)],
            sem,
        ).wait()


def _compute(rows_u, rows_v, accb, out_v, base_l):
    @plsc.parallel_loop(0, SUPER // 16, unroll=4)
    def _group(g):
        gvec = jnp.full((16,), 0, jnp.int32) + g
        for l in range(16):
            e = g * 16 + l
            ps = []
            for k in range(4):
                wu = rows_u[e, pl.ds(k * 16, 16)]
                wv = rows_v[e, pl.ds(k * 16, 16)]
                ps.append(plsc.bitcast(wu, jnp.bfloat16)
                          * plsc.bitcast(wv, jnp.bfloat16))
            psum = (ps[0] + ps[1]) + (ps[2] + ps[3])
            a, b = plsc.unpack(psum, format=plsc.PackFormat.INTERLEAVED,
                               preferred_element_type=jnp.float32)
            accb[g, l, pl.ds(0, 16)] = a + b
        le = lax.iota(jnp.int32, 16)
        tots = [jnp.zeros((16,), jnp.float32) for _ in range(4)]
        for c in range(16):
            cvec = jnp.full((16,), c, jnp.int32)
            tots[c % 4] = tots[c % 4] + plsc.load_gather(accb, [gvec, le, cvec])
        out_v[pl.ds(base_l + g * 16, 16)] = (tots[0] + tots[1]) + (tots[2] + tots[3])


def _sc_body(hp_hbm, src_hbm, dst_hbm, out_hbm,
             idx_u, idx_v, ru0, ru1, rv0, rv1, accb, out_v, hp_sp, sem0, sem1):
    wid = lax.axis_index("s") * NC + lax.axis_index("c")
    wbase = wid * EW

    sid = lax.axis_index("s")
    sbase = jnp.minimum(sid * 624, N_NODES_C - 640)
    pltpu.sync_copy(hp_hbm.at[pl.ds(sbase, 640)], hp_sp.at[pl.ds(sbase, 640)])
    plsc.subcore_barrier()

    pltpu.sync_copy(src_hbm.at[pl.ds(wbase, EW)], idx_u)
    pltpu.sync_copy(dst_hbm.at[pl.ds(wbase, EW)], idx_v)

    def base_of(s):
        return jnp.minimum(s * SUPER, LAST_BASE)

    def fire_s(s, ru, rv, sem):
        b = base_of(s)
        _fire(hp_sp, idx_u.at[pl.ds(b, SUPER)], ru, 0, sem)
        _fire(hp_sp, idx_v.at[pl.ds(b, SUPER)], rv, 0, sem)

    fire_s(0, ru0, rv0, sem0)

    @pl.loop(0, NSUPER // 2)
    def _pair(p):
        s0 = p * 2
        fire_s(s0 + 1, ru1, rv1, sem1)
        _drain(hp_sp, idx_u, ru0, sem0)
        _drain(hp_sp, idx_v, rv0, sem0)
        _compute(ru0, rv0, accb, out_v, base_of(s0))
        @pl.when(s0 + 2 < NSUPER)
        def _():
            fire_s(s0 + 2, ru0, rv0, sem0)
        _drain(hp_sp, idx_u, ru1, sem1)
        _drain(hp_sp, idx_v, rv1, sem1)
        _compute(ru1, rv1, accb, out_v, base_of(s0 + 1))

    pltpu.sync_copy(out_v, out_hbm.at[pl.ds(wbase, EW)])


@jax.jit
def _dot_pred(hp, src, dst):
    mesh = plsc.VectorSubcoreMesh(core_axis_name="c", subcore_axis_name="s")
    return pl.kernel(
        _sc_body,
        out_type=jax.ShapeDtypeStruct((N_EDGES_C,), jnp.float32),
        mesh=mesh,
        compiler_params=pltpu.CompilerParams(
            use_tc_tiling_on_sc=False, needs_layout_passes=False),
        scratch_types=[
            pltpu.VMEM((EW,), jnp.int32),
            pltpu.VMEM((EW,), jnp.int32),
            pltpu.VMEM((SUPER, DW), jnp.int32),
            pltpu.VMEM((SUPER, DW), jnp.int32),
            pltpu.VMEM((SUPER, DW), jnp.int32),
            pltpu.VMEM((SUPER, DW), jnp.int32),
            pltpu.VMEM((SUPER // 16, 16, ACCP), jnp.float32),
            pltpu.VMEM((EW,), jnp.float32),
            pltpu.VMEM_SHARED((N_NODES_C, DW), jnp.int32),
            pltpu.SemaphoreType.DMA,
            pltpu.SemaphoreType.DMA,
        ],
    )(hp, src, dst)


def kernel(h, edge_index):
    hb = h.astype(jnp.bfloat16)
    hp = lax.bitcast_convert_type(hb.reshape(N_NODES_C, DW, 2), jnp.int32)
    src = edge_index[0].astype(jnp.int32)
    dst = edge_index[1].astype(jnp.int32)
    score = _dot_pred(hp, src, dst)
    return score.reshape(N_EDGES_C, 1)


# speedup vs baseline: 1.6099x; 1.6099x over previous
import jax
import jax.numpy as jnp
from jax import lax
from jax.experimental import pallas as pl
from jax.experimental.pallas import tpu as pltpu
from jax.experimental.pallas import tpu_sc as plsc

N_NODES_C = 10000
N_EDGES_C = 320000
D = 128
DW = D // 2
ACCP = 17

NC = 2
NS = 16
NW = NC * NS
EW = N_EDGES_C // NW
SUPER = 128
GCH = 128
NSUPER = 80
LAST_BASE = EW - SUPER


def _fire(hp_hbm, idx, rows, base_l, sem):
    for k in range(SUPER // GCH):
        pltpu.async_copy(
            hp_hbm.at[idx.at[pl.ds(base_l + k * GCH, GCH)]],
            rows.at[pl.ds(k * GCH, GCH)],
            sem,
        )


def _drain(hp_hbm, idx, rows, sem):
    for k in range(SUPER // GCH):
        pltpu.make_async_copy(
            hp_hbm.at[idx.at[pl.ds(k * GCH, GCH)]],
            rows.at[pl.ds(k * GCH, GCH)],
            sem,
        ).wait()


def _compute(rows_u, rows_v, accb, out_v, base_l):
    @plsc.parallel_loop(0, SUPER // 16, unroll=2)
    def _group(g):
        gvec = jnp.full((16,), 0, jnp.int32) + g
        for l in range(16):
            e = g * 16 + l
            ps = []
            for k in range(4):
                wu = rows_u[e, pl.ds(k * 32, 32)]
                wv = rows_v[e, pl.ds(k * 32, 32)]
                ps.append(wu * wv)
            psum = (ps[0] + ps[1]) + (ps[2] + ps[3])
            a, b = plsc.unpack(psum, format=plsc.PackFormat.INTERLEAVED,
                               preferred_element_type=jnp.float32)
            accb[g, l, pl.ds(0, 16)] = a + b
        le = lax.iota(jnp.int32, 16)
        tots = [jnp.zeros((16,), jnp.float32) for _ in range(4)]
        for c in range(16):
            cvec = jnp.full((16,), c, jnp.int32)
            tots[c % 4] = tots[c % 4] + plsc.load_gather(accb, [gvec, le, cvec])
        out_v[pl.ds(base_l + g * 16, 16)] = (tots[0] + tots[1]) + (tots[2] + tots[3])


def _sc_body(hp_hbm, src_hbm, dst_hbm, out_hbm,
             idx_u, idx_v, ru0, ru1, rv0, rv1, accb, out_v, hp_sp, sem0, sem1):
    wid = lax.axis_index("s") * NC + lax.axis_index("c")
    wbase = wid * EW

    sid = lax.axis_index("s")
    sbase = jnp.minimum(sid * 624, N_NODES_C - 640)
    pltpu.sync_copy(hp_hbm.at[pl.ds(sbase, 640)], hp_sp.at[pl.ds(sbase, 640)])
    plsc.subcore_barrier()

    pltpu.sync_copy(src_hbm.at[pl.ds(wbase, EW)], idx_u)
    pltpu.sync_copy(dst_hbm.at[pl.ds(wbase, EW)], idx_v)

    def base_of(s):
        return jnp.minimum(s * SUPER, LAST_BASE)

    def fire_s(s, ru, rv, sem):
        b = base_of(s)
        _fire(hp_sp, idx_u.at[pl.ds(b, SUPER)], ru, 0, sem)
        _fire(hp_sp, idx_v.at[pl.ds(b, SUPER)], rv, 0, sem)

    fire_s(0, ru0, rv0, sem0)

    @pl.loop(0, NSUPER // 2)
    def _pair(p):
        s0 = p * 2
        fire_s(s0 + 1, ru1, rv1, sem1)
        _drain(hp_sp, idx_u, ru0, sem0)
        _drain(hp_sp, idx_v, rv0, sem0)
        _compute(ru0, rv0, accb, out_v, base_of(s0))
        @pl.when(s0 + 2 < NSUPER)
        def _():
            fire_s(s0 + 2, ru0, rv0, sem0)
        _drain(hp_sp, idx_u, ru1, sem1)
        _drain(hp_sp, idx_v, rv1, sem1)
        _compute(ru1, rv1, accb, out_v, base_of(s0 + 1))

    pltpu.sync_copy(out_v, out_hbm.at[pl.ds(wbase, EW)])


@jax.jit
def _dot_pred(hp, src, dst):
    mesh = plsc.VectorSubcoreMesh(core_axis_name="c", subcore_axis_name="s")
    return pl.kernel(
        _sc_body,
        out_type=jax.ShapeDtypeStruct((N_EDGES_C,), jnp.float32),
        mesh=mesh,
        compiler_params=pltpu.CompilerParams(
            use_tc_tiling_on_sc=False, needs_layout_passes=False),
        scratch_types=[
            pltpu.VMEM((EW,), jnp.int32),
            pltpu.VMEM((EW,), jnp.int32),
            pltpu.VMEM((SUPER, D), jnp.bfloat16),
            pltpu.VMEM((SUPER, D), jnp.bfloat16),
            pltpu.VMEM((SUPER, D), jnp.bfloat16),
            pltpu.VMEM((SUPER, D), jnp.bfloat16),
            pltpu.VMEM((SUPER // 16, 16, ACCP), jnp.float32),
            pltpu.VMEM((EW,), jnp.float32),
            pltpu.VMEM_SHARED((N_NODES_C, D), jnp.bfloat16),
            pltpu.SemaphoreType.DMA,
            pltpu.SemaphoreType.DMA,
        ],
    )(hp, src, dst)


def kernel(h, edge_index):
    hp = h.astype(jnp.bfloat16)
    src = edge_index[0].astype(jnp.int32)
    dst = edge_index[1].astype(jnp.int32)
    score = _dot_pred(hp, src, dst)
    return score.reshape(N_EDGES_C, 1)
